# baseline (speedup 1.0000x reference)
import functools

import jax
import jax.numpy as jnp
from jax import lax
from jax.experimental import pallas as pl
from jax.experimental.pallas import tpu as pltpu
from jax.experimental.pallas import tpu_sc as plsc

_CR = 16


def _make_sc_kernel(rows, ch, n):
    info = plsc.get_sparse_core_info()
    nw = info.num_cores * info.num_subcores
    chunk = rows // nw
    nsub = chunk // _CR
    mesh = plsc.VectorSubcoreMesh(core_axis_name="c", subcore_axis_name="s")

    @functools.partial(
        pl.kernel,
        mesh=mesh,
        out_type=jax.ShapeDtypeStruct((rows, n), jnp.float32),
        scratch_types=[
            pltpu.VMEM((n,), jnp.int32),
            pltpu.VMEM((_CR, ch), jnp.float32),
            pltpu.VMEM((_CR, n), jnp.float32),
        ],
        compiler_params=pltpu.CompilerParams(needs_layout_passes=False),
    )
    def k(x_hbm, idx_hbm, out_hbm, idx_v, buf, obuf):
        wid = lax.axis_index("s") * info.num_cores + lax.axis_index("c")
        base = wid * chunk
        pltpu.sync_copy(idx_hbm, idx_v)

        def step(i, carry):
            row0 = base + i * _CR
            pltpu.sync_copy(x_hbm.at[pl.ds(row0, _CR), :], buf)
            for r in range(_CR):
                for g in range(n // 16):
                    cols = idx_v[pl.ds(g * 16, 16)]
                    rr = jnp.full((16,), r, jnp.int32)
                    v = plsc.load_gather(buf, [rr, cols])
                    obuf[r, pl.ds(g * 16, 16)] = v
            pltpu.sync_copy(obuf, out_hbm.at[pl.ds(row0, _CR), :])
            return carry

        lax.fori_loop(0, nsub, step, 0)

    return k


def kernel(x, indices):
    b, s, ch = x.shape
    n = indices.shape[0]
    rows = b * s
    x2 = x.reshape(rows, ch)
    out = _make_sc_kernel(rows, ch, n)(x2, indices)
    return out.reshape(b, s, n)


# speedup vs baseline: 1.5878x; 1.5878x over previous
import functools

import jax
import jax.numpy as jnp
from jax import lax
from jax.experimental import pallas as pl
from jax.experimental.pallas import tpu as pltpu
from jax.experimental.pallas import tpu_sc as plsc

_ROWS_TC = 1024
_CR = 16
_SC_ROWS = 5120


def _tc_body(idx_ref, x_ref, o_ref):
    ch, n = x_ref.shape[1], o_ref.shape[1]
    c = jax.lax.broadcasted_iota(jnp.int32, (ch, n), 0)
    sel = (c == idx_ref[:][None, :]).astype(jnp.float32)
    o_ref[:] = jnp.dot(x_ref[:], sel, preferred_element_type=jnp.float32)


def _tc_call(x2, indices, row_lo, row_hi):
    rows = row_hi - row_lo
    n = indices.shape[0]
    ch = x2.shape[1]
    grid = rows // _ROWS_TC
    return pl.pallas_call(
        _tc_body,
        grid=(grid,),
        in_specs=[
            pl.BlockSpec((n,), lambda i: (0,)),
            pl.BlockSpec((_ROWS_TC, ch), lambda i: (i + row_lo // _ROWS_TC, 0)),
        ],
        out_specs=pl.BlockSpec((_ROWS_TC, n), lambda i: (i, 0)),
        out_shape=jax.ShapeDtypeStruct((rows, n), x2.dtype),
    )(indices, x2)


def _make_sc_kernel(rows_total, ch, n, row_lo, sc_rows):
    info = plsc.get_sparse_core_info()
    nw = info.num_cores * info.num_subcores
    chunk = sc_rows // nw
    nsub = chunk // _CR
    mesh = plsc.VectorSubcoreMesh(core_axis_name="c", subcore_axis_name="s")

    @functools.partial(
        pl.kernel,
        mesh=mesh,
        out_type=jax.ShapeDtypeStruct((sc_rows, n), jnp.float32),
        scratch_types=[
            pltpu.VMEM((n,), jnp.int32),
            pltpu.VMEM((_CR, ch), jnp.float32),
            pltpu.VMEM((_CR, n), jnp.float32),
        ],
        compiler_params=pltpu.CompilerParams(needs_layout_passes=False),
    )
    def k(x_hbm, idx_hbm, out_hbm, idx_v, buf, obuf):
        wid = lax.axis_index("s") * info.num_cores + lax.axis_index("c")
        base = wid * chunk
        pltpu.sync_copy(idx_hbm, idx_v)

        def step(i, carry):
            row0 = base + i * _CR
            pltpu.sync_copy(x_hbm.at[pl.ds(row_lo + row0, _CR), :], buf)
            for r in range(_CR):
                for g in range(n // 16):
                    cols = idx_v[pl.ds(g * 16, 16)]
                    rr = jnp.full((16,), r, jnp.int32)
                    v = plsc.load_gather(buf, [rr, cols])
                    obuf[r, pl.ds(g * 16, 16)] = v
            pltpu.sync_copy(obuf, out_hbm.at[pl.ds(row0, _CR), :])
            return carry

        lax.fori_loop(0, nsub, step, 0)

    return k


def kernel(x, indices):
    b, s, ch = x.shape
    n = indices.shape[0]
    rows = b * s
    x2 = x.reshape(rows, ch)
    r_tc = rows - _SC_ROWS
    out_tc = _tc_call(x2, indices, 0, r_tc)
    out_sc = _make_sc_kernel(rows, ch, n, r_tc, _SC_ROWS)(x2, indices)
    out = jnp.concatenate([out_tc, out_sc], axis=0)
    return out.reshape(b, s, n)
